# baseline (speedup 1.0000x reference)
import functools

import jax
import jax.numpy as jnp
from jax import lax
from jax.experimental import pallas as pl
from jax.experimental.pallas import tpu as pltpu
from jax.experimental.pallas import tpu_sc as plsc

N_NODES = 10000
N_EDGES = 320000
D = 128
W = D // 2
L = 16
NW = 32
EDGES_PER_W = N_EDGES // NW
C = 80
N_CHUNKS = EDGES_PER_W // C
N_PAIRS = N_CHUNKS // 2


@functools.lru_cache(maxsize=1)
def _build_score_kernel():
    mesh = plsc.VectorSubcoreMesh(core_axis_name="c", subcore_axis_name="s")

    @functools.partial(
        pl.kernel,
        mesh=mesh,
        compiler_params=pltpu.CompilerParams(needs_layout_passes=False,
                                             use_tc_tiling_on_sc=False),
        out_type=jax.ShapeDtypeStruct((N_EDGES,), jnp.float32),
        scratch_types=[
            pltpu.VMEM((EDGES_PER_W,), jnp.int32),
            pltpu.VMEM((EDGES_PER_W,), jnp.int32),
            pltpu.VMEM((2, C, W), jnp.int32),
            pltpu.VMEM((2, C, W), jnp.int32),
            pltpu.VMEM((EDGES_PER_W,), jnp.float32),
            pltpu.SemaphoreType.DMA((2,)),
            pltpu.SemaphoreType.DMA((2,)),
        ],
    )
    def _score_kernel(h_hbm, i_hbm, edges_hbm, out_hbm,
                      idx_u, idx_v, u_rows, v_rows, outs, sem_u, sem_v):
        wid = lax.axis_index("s") * 2 + lax.axis_index("c")
        base = wid * EDGES_PER_W
        pltpu.sync_copy(edges_hbm.at[pl.ds(base, EDGES_PER_W)], idx_u)
        pltpu.sync_copy(edges_hbm.at[pl.ds(N_EDGES + base, EDGES_PER_W)], idx_v)

        def start_gathers(k, b):
            pltpu.async_copy(h_hbm.at[idx_u.at[pl.ds(k * C, C)]],
                             u_rows.at[b], sem_u.at[b])
            pltpu.async_copy(i_hbm.at[idx_v.at[pl.ds(k * C, C)]],
                             v_rows.at[b], sem_v.at[b])

        def wait_gathers(b):
            pltpu.make_async_copy(h_hbm.at[idx_u.at[pl.ds(0, C)]],
                                  u_rows.at[b], sem_u.at[b]).wait()
            pltpu.make_async_copy(i_hbm.at[idx_v.at[pl.ds(0, C)]],
                                  v_rows.at[b], sem_v.at[b]).wait()

        lanes = lax.iota(jnp.int32, L)

        def compute_chunk(k, b):
            ub = u_rows.at[b]
            vb = v_rows.at[b]
            for e0 in range(0, C, L):
                rows = e0 + lanes

                def w_body(wb, acc, rows=rows, ub=ub, vb=vb):
                    acc8 = None
                    for j in range(8):
                        cols = (lanes + (wb * 8 + j)) & (W - 1)
                        ug = plsc.load_gather(ub, [rows, cols])
                        vg = plsc.load_gather(vb, [rows, cols])
                        prod = (plsc.bitcast(ug, jnp.bfloat16)
                                * plsc.bitcast(vg, jnp.bfloat16))
                        acc8 = prod if acc8 is None else acc8 + prod
                    pa, pb = plsc.unpack(acc8,
                                         format=plsc.PackFormat.INTERLEAVED)
                    return acc + pa + pb

                acc = lax.fori_loop(0, W // 8, w_body,
                                    jnp.zeros((L,), jnp.float32))
                outs[pl.ds(k * C + e0, L)] = acc

        start_gathers(0, 0)
        start_gathers(1, 1)

        def pair_body(p, carry):
            k0 = p * 2
            for b in range(2):
                k = k0 + b
                wait_gathers(b)
                compute_chunk(k, b)
                nxt = k + 2

                @pl.when(nxt < N_CHUNKS)
                def _():
                    start_gathers(nxt, b)

            return carry

        lax.fori_loop(0, N_PAIRS, pair_body, 0)

        wait_gathers(0)
        compute_chunk(N_CHUNKS - 1, 0)

        pltpu.sync_copy(outs, out_hbm.at[pl.ds(base, EDGES_PER_W)])

    return _score_kernel


def _pack_table(t):
    def rne16(x):
        u = lax.bitcast_convert_type(x, jnp.uint32)
        return (u + jnp.uint32(0x7FFF) + ((u >> 16) & jnp.uint32(1))) >> 16

    lo = rne16(t[:, :W])
    hi = rne16(t[:, W:])
    return lax.bitcast_convert_type((hi << 16) | lo, jnp.int32)


def kernel(h_new_P, i_embed, edge_index):
    edges = edge_index.astype(jnp.int32).reshape(-1)
    score = _build_score_kernel()(
        _pack_table(h_new_P), _pack_table(i_embed), edges)
    return score.reshape(N_EDGES, 1)


# speedup vs baseline: 1.2748x; 1.2748x over previous
import functools

import jax
import jax.numpy as jnp
from jax import lax
from jax.experimental import pallas as pl
from jax.experimental.pallas import tpu as pltpu
from jax.experimental.pallas import tpu_sc as plsc

N_NODES = 10000
N_EDGES = 320000
D = 128
W = D // 2
L = 16
NW = 32
EDGES_PER_W = N_EDGES // NW
C = 80
N_CHUNKS = EDGES_PER_W // C
N_PAIRS = N_CHUNKS // 2


@functools.lru_cache(maxsize=1)
def _build_score_kernel():
    mesh = plsc.VectorSubcoreMesh(core_axis_name="c", subcore_axis_name="s")

    @functools.partial(
        pl.kernel,
        mesh=mesh,
        compiler_params=pltpu.CompilerParams(needs_layout_passes=False,
                                             use_tc_tiling_on_sc=False),
        out_type=jax.ShapeDtypeStruct((N_EDGES,), jnp.float32),
        scratch_types=[
            pltpu.VMEM((EDGES_PER_W,), jnp.int32),
            pltpu.VMEM((EDGES_PER_W,), jnp.int32),
            pltpu.VMEM((2, C, W), jnp.int32),
            pltpu.VMEM((2, C, W), jnp.int32),
            pltpu.VMEM((EDGES_PER_W,), jnp.float32),
            pltpu.SemaphoreType.DMA((2,)),
            pltpu.SemaphoreType.DMA((2,)),
        ],
    )
    def _score_kernel(h_hbm, i_hbm, edges_hbm, out_hbm,
                      idx_u, idx_v, u_rows, v_rows, outs, sem_u, sem_v):
        wid = lax.axis_index("s") * 2 + lax.axis_index("c")
        base = wid * EDGES_PER_W
        pltpu.sync_copy(edges_hbm.at[pl.ds(base, EDGES_PER_W)], idx_u)
        pltpu.sync_copy(edges_hbm.at[pl.ds(N_EDGES + base, EDGES_PER_W)], idx_v)

        def start_gathers(k, b):
            pltpu.async_copy(h_hbm.at[idx_u.at[pl.ds(k * C, C)]],
                             u_rows.at[b], sem_u.at[b])
            pltpu.async_copy(i_hbm.at[idx_v.at[pl.ds(k * C, C)]],
                             v_rows.at[b], sem_v.at[b])

        def wait_gathers(b):
            pltpu.make_async_copy(h_hbm.at[idx_u.at[pl.ds(0, C)]],
                                  u_rows.at[b], sem_u.at[b]).wait()
            pltpu.make_async_copy(i_hbm.at[idx_v.at[pl.ds(0, C)]],
                                  v_rows.at[b], sem_v.at[b]).wait()

        lanes = lax.iota(jnp.int32, L)

        def compute_chunk(k, b):
            ub = u_rows.at[b]
            vb = v_rows.at[b]
            for e0 in range(0, C, L):
                rows = e0 + lanes

                def w_body(wb, acc, rows=rows, ub=ub, vb=vb):
                    acc8 = None
                    for j in range(8):
                        cols = (lanes + (wb * 8 + j)) & (W - 1)
                        ug = plsc.load_gather(ub, [rows, cols])
                        vg = plsc.load_gather(vb, [rows, cols])
                        prod = (plsc.bitcast(ug, jnp.bfloat16)
                                * plsc.bitcast(vg, jnp.bfloat16))
                        acc8 = prod if acc8 is None else acc8 + prod
                    pa, pb = plsc.unpack(acc8,
                                         format=plsc.PackFormat.INTERLEAVED)
                    return acc + pa + pb

                acc = lax.fori_loop(0, W // 8, w_body,
                                    jnp.zeros((L,), jnp.float32))
                outs[pl.ds(k * C + e0, L)] = acc

        start_gathers(0, 0)
        start_gathers(1, 1)

        def pair_body(p, carry):
            k0 = p * 2
            for b in range(2):
                k = k0 + b
                wait_gathers(b)
                compute_chunk(k, b)
                nxt = k + 2

                @pl.when(nxt < N_CHUNKS)
                def _():
                    start_gathers(nxt, b)

            return carry

        lax.fori_loop(0, N_PAIRS, pair_body, 0)

        wait_gathers(0)
        compute_chunk(N_CHUNKS - 1, 0)

        pltpu.sync_copy(outs, out_hbm.at[pl.ds(base, EDGES_PER_W)])

    return _score_kernel


def _pack_table(t):
    bits = lax.bitcast_convert_type(t, jnp.uint32)

    def rne16(u):
        return (u + jnp.uint32(0x7FFF) + ((u >> 16) & jnp.uint32(1))) >> 16

    lo = rne16(bits[:, :W])
    hi = rne16(bits[:, W:])
    return lax.bitcast_convert_type((hi << 16) | lo, jnp.int32)


def kernel(h_new_P, i_embed, edge_index):
    edges = edge_index.astype(jnp.int32).reshape(-1)
    score = _build_score_kernel()(
        _pack_table(h_new_P), _pack_table(i_embed), edges)
    return score.reshape(N_EDGES, 1)
